# baseline (speedup 1.0000x reference)
import functools

import jax
import jax.numpy as jnp
from jax import lax
from jax.experimental import pallas as pl
from jax.experimental.pallas import tpu as pltpu
from jax.experimental.pallas import tpu_sc as plsc

ROWS, COLS = 4096, 200
D_MODEL = 64
B = ROWS * COLS

_info = plsc.get_sparse_core_info()
_NC, _NS = _info.num_cores, _info.num_subcores
NW = _NC * _NS
BPW = B // NW
BBLK = ROWS // NW
SBLK = 2
TPB = SBLK * BBLK
NBLK = COLS // SBLK
LT = D_MODEL // 8
OBUF_N = SBLK * D_MODEL * BBLK

_mesh = plsc.VectorSubcoreMesh(core_axis_name="c", subcore_axis_name="s")


@functools.partial(
    pl.kernel,
    mesh=_mesh,
    out_type=jax.ShapeDtypeStruct((B * D_MODEL,), jnp.float32),
    compiler_params=pltpu.CompilerParams(
        use_tc_tiling_on_sc=False, needs_layout_passes=False),
    scratch_types=[
        pltpu.VMEM((BPW,), jnp.int32),
        pltpu.VMEM((TPB,), jnp.int32),
        pltpu.VMEM((TPB,), jnp.int32),
        pltpu.VMEM((TPB, D_MODEL), jnp.float32),
        pltpu.VMEM((TPB, D_MODEL), jnp.float32),
        pltpu.VMEM((OBUF_N,), jnp.float32),
        pltpu.VMEM((OBUF_N,), jnp.float32),
        pltpu.SemaphoreType.DMA,
        pltpu.SemaphoreType.DMA,
        pltpu.SemaphoreType.DMA,
        pltpu.SemaphoreType.DMA,
    ],
)
def _emb_lookup(idx_hbm, table_hbm, out_hbm, idx_all, gl0, gl1,
                rows0, rows1, obuf0, obuf1, sg0, sg1, ss0, ss1):
    wid = lax.axis_index("s") * _NC + lax.axis_index("c")
    gl = (gl0, gl1)
    rows = (rows0, rows1)
    obuf = (obuf0, obuf1)
    sg = (sg0, sg1)
    ss = (ss0, ss1)

    iota = lax.iota(jnp.int32, 16)
    iota_cols = iota * COLS
    dvec128 = [(iota + 16 * k) * 128 for k in range(4)]

    pltpu.sync_copy(idx_hbm.at[pl.ds(wid * BPW, BPW)], idx_all)

    def build_glist(c, p):
        for s_local in range(SBLK):
            s = c * SBLK + s_local

            @pl.loop(0, BBLK, step=16)
            def _(b0):
                addr = iota_cols + (b0 * COLS + s)
                gl[p][pl.ds(s_local * BBLK + b0, 16)] = plsc.load_gather(
                    idx_all, [addr])

    def start_gather(c, p):
        for j in range(TPB // 128):
            pltpu.async_copy(
                table_hbm.at[gl[p].at[pl.ds(j * 128, 128)]],
                rows[p].at[pl.ds(j * 128, 128)],
                sg[p],
            )

    def drain_gather(p):
        for j in range(TPB // 128):
            pltpu.make_async_copy(
                table_hbm.at[gl[p].at[pl.ds(j * 128, 128)]],
                rows[p].at[pl.ds(j * 128, 128)],
                sg[p],
            ).wait()

    def transpose_block(p):
        for s_local in range(SBLK):
            base_s = s_local * (D_MODEL * BBLK)

            @pl.loop(0, BBLK, unroll=2)
            def _(b):
                t = s_local * BBLK + b
                for k in range(4):
                    v = rows[p][t, pl.ds(k * 16, 16)]
                    plsc.store_scatter(obuf[p], [dvec128[k] + (base_s + b)], v)

    def start_store(c, p):
        for s_local in range(SBLK):
            s = c * SBLK + s_local
            for lt in range(LT):
                src = obuf[p].at[pl.ds(
                    s_local * D_MODEL * BBLK + lt * 8 * BBLK, 8 * BBLK)]
                dst = out_hbm.at[pl.ds(
                    ((s * LT + lt) * NW + wid) * 8 * BBLK, 8 * BBLK)]
                pltpu.async_copy(src, dst, ss[p])

    def wait_store(p):
        for _ in range(SBLK * LT):
            pltpu.make_async_copy(
                obuf[p].at[pl.ds(0, 8 * BBLK)],
                out_hbm.at[pl.ds(0, 8 * BBLK)],
                ss[p],
            ).wait()

    for p in range(2):
        build_glist(p, p)
        start_gather(p, p)
    for p in range(2):
        drain_gather(p)
        transpose_block(p)
        build_glist(p + 2, p)
        start_gather(p + 2, p)
        start_store(p, p)

    @pl.loop(2, NBLK, step=2)
    def _main(i):
        for p in range(2):
            c = i + p
            drain_gather(p)
            wait_store(p)
            transpose_block(p)

            @pl.when(c + 2 < NBLK)
            def _():
                build_glist(c + 2, p)
                start_gather(c + 2, p)

            start_store(c, p)

    for p in range(2):
        wait_store(p)


def kernel(x, table):
    idx = x.astype(jnp.int32).reshape(B)
    out_flat = _emb_lookup(idx, table)
    out5 = out_flat.reshape(COLS, LT, NW, 8, BBLK)
    return jnp.transpose(out5, (2, 4, 0, 1, 3)).reshape(ROWS, COLS, D_MODEL)


# speedup vs baseline: 1.2161x; 1.2161x over previous
import functools

import jax
import jax.numpy as jnp
from jax import lax
from jax.experimental import pallas as pl
from jax.experimental.pallas import tpu as pltpu
from jax.experimental.pallas import tpu_sc as plsc

ROWS, COLS = 4096, 200
D_MODEL = 64
B = ROWS * COLS

_info = plsc.get_sparse_core_info()
_NC, _NS = _info.num_cores, _info.num_subcores
NW = _NC * _NS
BPW = B // NW
BBLK = ROWS // NW
SBLK = 2
TPB = SBLK * BBLK
NBLK = COLS // SBLK
LT = D_MODEL // 8
OBUF_N = SBLK * D_MODEL * BBLK

_mesh = plsc.VectorSubcoreMesh(core_axis_name="c", subcore_axis_name="s")


@functools.partial(
    pl.kernel,
    mesh=_mesh,
    out_type=jax.ShapeDtypeStruct((B * D_MODEL,), jnp.float32),
    compiler_params=pltpu.CompilerParams(
        use_tc_tiling_on_sc=False, needs_layout_passes=False),
    scratch_types=[
        pltpu.VMEM((BPW,), jnp.int32),
        pltpu.VMEM((TPB,), jnp.int32),
        pltpu.VMEM((TPB,), jnp.int32),
        pltpu.VMEM((TPB, D_MODEL), jnp.float32),
        pltpu.VMEM((TPB, D_MODEL), jnp.float32),
        pltpu.VMEM((OBUF_N,), jnp.float32),
        pltpu.VMEM((OBUF_N,), jnp.float32),
        pltpu.SemaphoreType.DMA,
        pltpu.SemaphoreType.DMA,
        pltpu.SemaphoreType.DMA,
        pltpu.SemaphoreType.DMA,
    ],
)
def _emb_lookup(idx_hbm, table_hbm, out_hbm, idx_all, gl0, gl1,
                rows0, rows1, obuf0, obuf1, sg0, sg1, ss0, ss1):
    wid = lax.axis_index("s") * _NC + lax.axis_index("c")
    gl = (gl0, gl1)
    rows = (rows0, rows1)
    obuf = (obuf0, obuf1)
    sg = (sg0, sg1)
    ss = (ss0, ss1)

    iota = lax.iota(jnp.int32, 16)
    iota_cols = iota * COLS
    dvec128 = [(iota + 16 * k) * 128 for k in range(4)]

    pltpu.sync_copy(idx_hbm.at[pl.ds(wid * BPW, BPW)], idx_all)

    def build_glist(c, p):
        for s_local in range(SBLK):
            s = c * SBLK + s_local

            @pl.loop(0, BBLK, step=16)
            def _(b0):
                addr = iota_cols + (b0 * COLS + s)
                gl[p][pl.ds(s_local * BBLK + b0, 16)] = plsc.load_gather(
                    idx_all, [addr])

    def start_gather(c, p):
        for j in range(TPB // 128):
            pltpu.async_copy(
                table_hbm.at[gl[p].at[pl.ds(j * 128, 128)]],
                rows[p].at[pl.ds(j * 128, 128)],
                sg[p],
            )

    def drain_gather(p):
        for j in range(TPB // 128):
            pltpu.make_async_copy(
                table_hbm.at[gl[p].at[pl.ds(j * 128, 128)]],
                rows[p].at[pl.ds(j * 128, 128)],
                sg[p],
            ).wait()

    def transpose_block(p):
        for s_local in range(SBLK):
            base_s = s_local * (D_MODEL * BBLK)

            @plsc.parallel_loop(0, BBLK, unroll=4)
            def _(b):
                t = s_local * BBLK + b
                for k in range(4):
                    v = rows[p][t, pl.ds(k * 16, 16)]
                    plsc.store_scatter(obuf[p], [dvec128[k] + (base_s + b)], v)

    def start_store(c, p):
        for s_local in range(SBLK):
            s = c * SBLK + s_local
            for lt in range(LT):
                src = obuf[p].at[pl.ds(
                    s_local * D_MODEL * BBLK + lt * 8 * BBLK, 8 * BBLK)]
                dst = out_hbm.at[pl.ds(
                    ((s * LT + lt) * NW + wid) * 8 * BBLK, 8 * BBLK)]
                pltpu.async_copy(src, dst, ss[p])

    def wait_store(p):
        for _ in range(SBLK * LT):
            pltpu.make_async_copy(
                obuf[p].at[pl.ds(0, 8 * BBLK)],
                out_hbm.at[pl.ds(0, 8 * BBLK)],
                ss[p],
            ).wait()

    for p in range(2):
        build_glist(p, p)
        start_gather(p, p)
    for p in range(2):
        drain_gather(p)
        transpose_block(p)
        build_glist(p + 2, p)
        start_gather(p + 2, p)
        start_store(p, p)

    @pl.loop(2, NBLK, step=2)
    def _main(i):
        for p in range(2):
            c = i + p
            drain_gather(p)
            wait_store(p)
            transpose_block(p)

            @pl.when(c + 2 < NBLK)
            def _():
                build_glist(c + 2, p)
                start_gather(c + 2, p)

            start_store(c, p)

    for p in range(2):
        wait_store(p)


def kernel(x, table):
    idx = x.astype(jnp.int32).reshape(B)
    out_flat = _emb_lookup(idx, table)
    out5 = out_flat.reshape(COLS, LT, NW, 8, BBLK)
    return jnp.transpose(out5, (2, 4, 0, 1, 3)).reshape(ROWS, COLS, D_MODEL)


# speedup vs baseline: 2.1341x; 1.7548x over previous
import functools

import jax
import jax.numpy as jnp
from jax import lax
from jax.experimental import pallas as pl
from jax.experimental.pallas import tpu as pltpu
from jax.experimental.pallas import tpu_sc as plsc

ROWS, COLS = 4096, 200
D_MODEL = 64
B = ROWS * COLS

_info = plsc.get_sparse_core_info()
_NC, _NS = _info.num_cores, _info.num_subcores
NW = _NC * _NS
BPW = B // NW
BBLK = ROWS // NW
SBLK = 2
TPB = SBLK * BBLK
NBLK = COLS // SBLK
LT = D_MODEL // 8
COLSP = COLS + 1
OBW = BBLK + 1

_mesh = plsc.VectorSubcoreMesh(core_axis_name="c", subcore_axis_name="s")


@functools.partial(
    pl.kernel,
    mesh=_mesh,
    out_type=jax.ShapeDtypeStruct((COLS, LT, NW, 8, BBLK), jnp.float32),
    compiler_params=pltpu.CompilerParams(
        use_tc_tiling_on_sc=False, needs_layout_passes=False),
    scratch_types=[
        pltpu.VMEM((BBLK, COLSP), jnp.int32),
        pltpu.VMEM((TPB,), jnp.int32),
        pltpu.VMEM((TPB,), jnp.int32),
        pltpu.VMEM((TPB, D_MODEL), jnp.float32),
        pltpu.VMEM((TPB, D_MODEL), jnp.float32),
        pltpu.VMEM((SBLK * D_MODEL, OBW), jnp.float32),
        pltpu.VMEM((SBLK * D_MODEL, OBW), jnp.float32),
        pltpu.SemaphoreType.DMA,
        pltpu.SemaphoreType.DMA,
        pltpu.SemaphoreType.DMA,
        pltpu.SemaphoreType.DMA,
    ],
)
def _emb_lookup(x_hbm, table_hbm, out_hbm, idx_all, gl0, gl1,
                rows0, rows1, obuf0, obuf1, sg0, sg1, ss0, ss1):
    wid = lax.axis_index("s") * _NC + lax.axis_index("c")
    gl = (gl0, gl1)
    rows = (rows0, rows1)
    obuf = (obuf0, obuf1)
    sg = (sg0, sg1)
    ss = (ss0, ss1)

    iota = lax.iota(jnp.int32, 16)
    dvec = [iota + 16 * k for k in range(4)]

    pltpu.sync_copy(x_hbm.at[pl.ds(wid * BBLK, BBLK)],
                    idx_all.at[:, pl.ds(0, COLS)])

    def build_glist(c, p):
        for s_local in range(SBLK):
            s = c * SBLK + s_local

            @plsc.parallel_loop(0, BBLK, step=16)
            def _(b0):
                bvec = iota + b0
                gl[p][pl.ds(s_local * BBLK + b0, 16)] = plsc.load_gather(
                    idx_all, [bvec, jnp.full((16,), s, jnp.int32)])

    def start_gather(c, p):
        for j in range(TPB // 128):
            pltpu.async_copy(
                table_hbm.at[gl[p].at[pl.ds(j * 128, 128)]],
                rows[p].at[pl.ds(j * 128, 128)],
                sg[p],
            )

    def drain_gather(p):
        for j in range(TPB // 128):
            pltpu.make_async_copy(
                table_hbm.at[gl[p].at[pl.ds(j * 128, 128)]],
                rows[p].at[pl.ds(j * 128, 128)],
                sg[p],
            ).wait()

    def transpose_block(p):
        for s_local in range(SBLK):
            rvec = [dvec[k] + s_local * D_MODEL for k in range(4)]

            @plsc.parallel_loop(0, BBLK, unroll=4)
            def _(b):
                t = s_local * BBLK + b
                bvec = jnp.full((16,), b, jnp.int32)
                for k in range(4):
                    v = rows[p][t, pl.ds(k * 16, 16)]
                    plsc.store_scatter(obuf[p], [rvec[k], bvec], v)

    def start_store(c, p):
        for s_local in range(SBLK):
            s = c * SBLK + s_local
            for lt in range(LT):
                src = obuf[p].at[pl.ds(s_local * D_MODEL + lt * 8, 8),
                                 pl.ds(0, BBLK)]
                pltpu.async_copy(src, out_hbm.at[s, lt, wid], ss[p])

    def wait_store(p):
        for _ in range(SBLK * LT):
            pltpu.make_async_copy(
                obuf[p].at[pl.ds(0, 8), pl.ds(0, BBLK)],
                out_hbm.at[0, 0, wid],
                ss[p],
            ).wait()

    for p in range(2):
        build_glist(p, p)
        start_gather(p, p)
    for p in range(2):
        drain_gather(p)
        transpose_block(p)
        build_glist(p + 2, p)
        start_gather(p + 2, p)
        start_store(p, p)

    @pl.loop(2, NBLK, step=2)
    def _main(i):
        for p in range(2):
            c = i + p
            drain_gather(p)
            wait_store(p)
            transpose_block(p)

            @pl.when(c + 2 < NBLK)
            def _():
                build_glist(c + 2, p)
                start_gather(c + 2, p)

            start_store(c, p)

    for p in range(2):
        wait_store(p)


def kernel(x, table):
    out5 = _emb_lookup(x.astype(jnp.int32), table)
    return jnp.transpose(out5, (2, 4, 0, 1, 3)).reshape(ROWS, COLS, D_MODEL)
